# Initial kernel scaffold; baseline (speedup 1.0000x reference)
import functools

import jax
import jax.numpy as jnp
from jax import lax
from jax.experimental import pallas as pl
from jax.experimental.pallas import tpu as pltpu
from jax.experimental.pallas import tpu_sc as plsc

_LIG_N = 2048
_REC_N = 10000
_LIG_E = 32768
_REC_E = 160000
_D = 16
_NW = 32


def _lrelu(x):
    return jnp.where(x >= 0, x, 0.01 * x)


def _sc_mesh():
    return plsc.VectorSubcoreMesh(core_axis_name="c", subcore_axis_name="s")


def _sc_gather(table, idx3, E, C, Lc):
    chunk = C * Lc

    @functools.partial(
        pl.kernel,
        out_type=jax.ShapeDtypeStruct((E, _D), jnp.float32),
        mesh=_sc_mesh(),
        scratch_types=[
            pltpu.VMEM((C, Lc), jnp.int32),
            pltpu.VMEM((chunk, _D), jnp.float32),
            pltpu.SemaphoreType.DMA,
        ],
        name="sc_gather",
    )
    def k(table_hbm, idx_hbm, out_hbm, idx_v, rows_v, sem):
        cid = lax.axis_index("c")
        sid = lax.axis_index("s")
        wid = sid * 2 + cid
        pltpu.sync_copy(idx_hbm.at[wid], idx_v)

        @pl.loop(0, C)
        def _issue(j):
            pltpu.async_copy(table_hbm.at[idx_v.at[j]],
                             rows_v.at[pl.ds(j * Lc, Lc)], sem)

        @pl.loop(0, C)
        def _drain(j):
            pltpu.make_async_copy(table_hbm.at[idx_v.at[j]],
                                  rows_v.at[pl.ds(j * Lc, Lc)], sem).wait()

        pltpu.sync_copy(rows_v, out_hbm.at[pl.ds(wid * chunk, chunk)])

    return k(table, idx3)


def _sc_scatter(vals, idx3, zeros, N, C, Lc):
    chunk = C * Lc
    rows = N // 16

    @functools.partial(
        pl.kernel,
        out_type=jax.ShapeDtypeStruct((2, N, _D), jnp.float32),
        mesh=_sc_mesh(),
        scratch_types=[
            pltpu.VMEM((C, Lc), jnp.int32),
            pltpu.VMEM((chunk, _D), jnp.float32),
            pltpu.VMEM_SHARED((N, _D), jnp.float32),
            pltpu.SemaphoreType.DMA,
        ],
        name="sc_scatter",
    )
    def k(vals_hbm, idx_hbm, zeros_hbm, out_hbm, idx_v, vals_v, shared, sem):
        cid = lax.axis_index("c")
        sid = lax.axis_index("s")
        wid = sid * 2 + cid
        base = sid * rows
        pltpu.sync_copy(zeros_hbm.at[pl.ds(base, rows)],
                        shared.at[pl.ds(base, rows)])
        pltpu.sync_copy(idx_hbm.at[wid], idx_v)
        pltpu.sync_copy(vals_hbm.at[pl.ds(wid * chunk, chunk)], vals_v)
        plsc.subcore_barrier()

        @pl.loop(0, C)
        def _scat(j):
            pltpu.sync_copy(vals_v.at[pl.ds(j * Lc, Lc)],
                            shared.at[idx_v.at[j]], add=True)

        plsc.subcore_barrier()
        pltpu.sync_copy(shared.at[pl.ds(base, rows)],
                        out_hbm.at[cid, pl.ds(base, rows)])

    return k(vals, idx3, zeros)


def _sc_count(idx3, ones, zeros, N, C, Lc):
    rows = N // 16

    @functools.partial(
        pl.kernel,
        out_type=jax.ShapeDtypeStruct((2, N, _D), jnp.float32),
        mesh=_sc_mesh(),
        scratch_types=[
            pltpu.VMEM((C, Lc), jnp.int32),
            pltpu.VMEM((Lc, _D), jnp.float32),
            pltpu.VMEM_SHARED((N, _D), jnp.float32),
            pltpu.SemaphoreType.DMA,
        ],
        name="sc_count",
    )
    def k(idx_hbm, ones_hbm, zeros_hbm, out_hbm, idx_v, ones_v, shared, sem):
        cid = lax.axis_index("c")
        sid = lax.axis_index("s")
        wid = sid * 2 + cid
        base = sid * rows
        pltpu.sync_copy(zeros_hbm.at[pl.ds(base, rows)],
                        shared.at[pl.ds(base, rows)])
        pltpu.sync_copy(idx_hbm.at[wid], idx_v)
        pltpu.sync_copy(ones_hbm, ones_v)
        plsc.subcore_barrier()

        @pl.loop(0, C)
        def _scat(j):
            pltpu.sync_copy(ones_v, shared.at[idx_v.at[j]], add=True)

        plsc.subcore_barrier()
        pltpu.sync_copy(shared.at[pl.ds(base, rows)],
                        out_hbm.at[cid, pl.ds(base, rows)])

    return k(idx3, ones, zeros)


def _tc_embed(x, W1, b1, W2, b2):
    N = x.shape[0]

    def body(x_ref, W1_ref, b1_ref, W2_ref, b2_ref, h_ref, g_ref):
        nf = jnp.dot(x_ref[...], W1_ref[...],
                     preferred_element_type=jnp.float32) + b1_ref[...]
        h = jnp.dot(_lrelu(nf), W2_ref[...],
                    preferred_element_type=jnp.float32) + b2_ref[...]
        h_ref[...] = h
        g_ref[...] = _lrelu(h)

    return pl.pallas_call(
        body,
        out_shape=[jax.ShapeDtypeStruct((N, _D), jnp.float32),
                   jax.ShapeDtypeStruct((N, _D), jnp.float32)],
    )(x, W1, b1.reshape(1, -1), W2, b2.reshape(1, -1))


def _tc_msg(gT, efT, WefT, bef2, B):
    E = efT.shape[1]

    def body(gT_ref, efT_ref, W_ref, b_ref, out_ref):
        Q = jnp.dot(W_ref[...], efT_ref[...],
                    preferred_element_type=jnp.float32) + b_ref[...]
        g = gT_ref[...]
        acc = g[0:1, :] * Q[0:_D, :]
        for i in range(1, _D):
            acc = acc + g[i:i + 1, :] * Q[i * _D:(i + 1) * _D, :]
        out_ref[...] = acc

    return pl.pallas_call(
        body,
        grid=(E // B,),
        in_specs=[
            pl.BlockSpec((_D, B), lambda i: (0, i)),
            pl.BlockSpec((_D, B), lambda i: (0, i)),
            pl.BlockSpec((_D * _D, _D), lambda i: (0, 0)),
            pl.BlockSpec((_D * _D, 1), lambda i: (0, 0)),
        ],
        out_specs=pl.BlockSpec((_D, B), lambda i: (0, i)),
        out_shape=jax.ShapeDtypeStruct((_D, E), jnp.float32),
    )(gT, efT, WefT, bef2)


def _tc_update(p0, p1, c0, c1, b_nn, W_out, b_out):
    N = p0.shape[0]

    def body(p0_ref, p1_ref, c0_ref, c1_ref, bn_ref, W_ref, bo_ref,
             h_ref, g_ref):
        s = p0_ref[...] + p1_ref[...]
        cnt = jnp.maximum(c0_ref[...] + c1_ref[...], 1.0)
        agg = s / cnt + bn_ref[...]
        h = jnp.dot(_lrelu(agg), W_ref[...],
                    preferred_element_type=jnp.float32) + bo_ref[...]
        h_ref[...] = h
        g_ref[...] = _lrelu(h)

    return pl.pallas_call(
        body,
        out_shape=[jax.ShapeDtypeStruct((N, _D), jnp.float32),
                   jax.ShapeDtypeStruct((N, _D), jnp.float32)],
    )(p0, p1, c0, c1, b_nn.reshape(1, -1), W_out, b_out.reshape(1, -1))


def _tc_interaction(lig_h, rec_h):
    L, R = lig_h.shape[0], rec_h.shape[0]
    BL = 256
    grid = (L // BL,)
    inv = 1.0 / (L * R)

    def body(l_ref, r_ref, op_ref, s_ref):
        i = pl.program_id(0)
        op_ref[...] = lax.dot_general(
            l_ref[...], r_ref[...], (((1,), (1,)), ((), ())),
            preferred_element_type=jnp.float32)
        sl = jnp.sum(l_ref[...], axis=0, keepdims=True)
        sr = jnp.sum(r_ref[...], axis=0, keepdims=True)
        part = jnp.sum(sl * sr) * inv

        @pl.when(i == 0)
        def _():
            s_ref[...] = jnp.zeros_like(s_ref)

        s_ref[...] = s_ref[...] + part

    op, s8 = pl.pallas_call(
        body,
        grid=grid,
        in_specs=[pl.BlockSpec((BL, _D), lambda i: (i, 0)),
                  pl.BlockSpec((R, _D), lambda i: (0, 0))],
        out_specs=[pl.BlockSpec((BL, R), lambda i: (i, 0)),
                   pl.BlockSpec((8, 128), lambda i: (0, 0))],
        out_shape=[jax.ShapeDtypeStruct((L, R), jnp.float32),
                   jax.ShapeDtypeStruct((8, 128), jnp.float32)],
    )(lig_h, rec_h)
    return op, s8[0, 0].reshape(1)


def _graph_chain(x, eattr, eidx, W_emb, b_emb, W_hid, b_hid,
                 W_ef, b_ef, b_nn, W_out, b_out, N, E, C, Lc, B):
    src = eidx[0].astype(jnp.int32).reshape(_NW, C, Lc)
    dst = eidx[1].astype(jnp.int32).reshape(_NW, C, Lc)
    efT = eattr.T
    zeros = jnp.zeros((N, _D), jnp.float32)
    ones = jnp.ones((Lc, _D), jnp.float32)

    h, g = _tc_embed(x, W_emb, b_emb, W_hid, b_hid)
    cparts = _sc_count(dst, ones, zeros, N, C, Lc)
    c0, c1 = cparts[0], cparts[1]

    for layer in range(W_ef.shape[0]):
        hg = _sc_gather(g, src, E, C, Lc)
        msgT = _tc_msg(hg.T, efT, W_ef[layer].T,
                       b_ef[layer].reshape(_D * _D, 1), B)
        parts = _sc_scatter(msgT.T, dst, zeros, N, C, Lc)
        h, g = _tc_update(parts[0], parts[1], c0, c1,
                          b_nn[layer], W_out[layer], b_out[layer])
    return h


def kernel(lig_x, rec_x, lig_edge_attr, rec_edge_attr, lig_edge_index,
           rec_edge_index, W_emb_lig, b_emb_lig, W_emb_rec, b_emb_rec,
           W_hid_lig, b_hid_lig, W_hid_rec, b_hid_rec,
           lig_W_ef, lig_b_ef, lig_b_nn, lig_W_out, lig_b_out,
           rec_W_ef, rec_b_ef, rec_b_nn, rec_W_out, rec_b_out):
    lig_h = _graph_chain(lig_x, lig_edge_attr, lig_edge_index,
                         W_emb_lig, b_emb_lig, W_hid_lig, b_hid_lig,
                         lig_W_ef, lig_b_ef, lig_b_nn, lig_W_out, lig_b_out,
                         _LIG_N, _LIG_E, C=8, Lc=128, B=4096)
    rec_h = _graph_chain(rec_x, rec_edge_attr, rec_edge_index,
                         W_emb_rec, b_emb_rec, W_hid_rec, b_hid_rec,
                         rec_W_ef, rec_b_ef, rec_b_nn, rec_W_out, rec_b_out,
                         _REC_N, _REC_E, C=40, Lc=125, B=6400)
    op, out = _tc_interaction(lig_h, rec_h)
    return (out, op)


# speedup vs baseline: 5.7722x; 5.7722x over previous
import functools

import jax
import jax.numpy as jnp
from jax import lax
from jax.experimental import pallas as pl
from jax.experimental.pallas import tpu as pltpu
from jax.experimental.pallas import tpu_sc as plsc

_LIG_N = 2048
_REC_N = 10000
_LIG_E = 32768
_REC_E = 160000
_D = 16
_NW = 32


def _lrelu(x):
    return jnp.where(x >= 0, x, 0.01 * x)


def _sc_mesh():
    return plsc.VectorSubcoreMesh(core_axis_name="c", subcore_axis_name="s")


_SC_PARAMS = pltpu.CompilerParams(use_tc_tiling_on_sc=False)


def _sc_gather(table, idx3, E, C, Lc):
    chunk = C * Lc

    @functools.partial(
        pl.kernel,
        out_type=jax.ShapeDtypeStruct((E, _D), jnp.float32),
        mesh=_sc_mesh(),
        scratch_types=[
            pltpu.VMEM((C, Lc), jnp.int32),
            pltpu.VMEM((chunk, _D), jnp.float32),
            pltpu.SemaphoreType.DMA,
        ],
        name="sc_gather",
        compiler_params=_SC_PARAMS,
    )
    def k(table_hbm, idx_hbm, out_hbm, idx_v, rows_v, sem):
        cid = lax.axis_index("c")
        sid = lax.axis_index("s")
        wid = sid * 2 + cid
        pltpu.sync_copy(idx_hbm.at[wid], idx_v)

        @pl.loop(0, C)
        def _issue(j):
            pltpu.async_copy(table_hbm.at[idx_v.at[j]],
                             rows_v.at[pl.ds(j * Lc, Lc)], sem)

        @pl.loop(0, C)
        def _drain(j):
            pltpu.make_async_copy(table_hbm.at[idx_v.at[j]],
                                  rows_v.at[pl.ds(j * Lc, Lc)], sem).wait()

        pltpu.sync_copy(rows_v, out_hbm.at[pl.ds(wid * chunk, chunk)])

    return k(table, idx3)


def _sc_scatter(vals, idx3, zeros, N, C, Lc):
    chunk = C * Lc
    rows = N // 16

    @functools.partial(
        pl.kernel,
        out_type=jax.ShapeDtypeStruct((2, N, _D), jnp.float32),
        mesh=_sc_mesh(),
        scratch_types=[
            pltpu.VMEM((C, Lc), jnp.int32),
            pltpu.VMEM((chunk, _D), jnp.float32),
            pltpu.VMEM_SHARED((N, _D), jnp.float32),
            pltpu.SemaphoreType.DMA,
        ],
        name="sc_scatter",
        compiler_params=_SC_PARAMS,
    )
    def k(vals_hbm, idx_hbm, zeros_hbm, out_hbm, idx_v, vals_v, shared, sem):
        cid = lax.axis_index("c")
        sid = lax.axis_index("s")
        wid = sid * 2 + cid
        base = sid * rows
        pltpu.sync_copy(zeros_hbm.at[pl.ds(base, rows)],
                        shared.at[pl.ds(base, rows)])
        pltpu.sync_copy(idx_hbm.at[wid], idx_v)
        pltpu.sync_copy(vals_hbm.at[pl.ds(wid * chunk, chunk)], vals_v)
        plsc.subcore_barrier()

        @pl.loop(0, C)
        def _scat(j):
            pltpu.sync_copy(vals_v.at[pl.ds(j * Lc, Lc)],
                            shared.at[idx_v.at[j]], add=True)

        plsc.subcore_barrier()
        pltpu.sync_copy(shared.at[pl.ds(base, rows)],
                        out_hbm.at[cid, pl.ds(base, rows)])

    return k(vals, idx3, zeros)


def _sc_count(idx3, ones, zeros, N, C, Lc):
    rows = N // 16

    @functools.partial(
        pl.kernel,
        out_type=jax.ShapeDtypeStruct((2, N, _D), jnp.float32),
        mesh=_sc_mesh(),
        scratch_types=[
            pltpu.VMEM((C, Lc), jnp.int32),
            pltpu.VMEM((Lc, _D), jnp.float32),
            pltpu.VMEM_SHARED((N, _D), jnp.float32),
            pltpu.SemaphoreType.DMA,
        ],
        name="sc_count",
        compiler_params=_SC_PARAMS,
    )
    def k(idx_hbm, ones_hbm, zeros_hbm, out_hbm, idx_v, ones_v, shared, sem):
        cid = lax.axis_index("c")
        sid = lax.axis_index("s")
        wid = sid * 2 + cid
        base = sid * rows
        pltpu.sync_copy(zeros_hbm.at[pl.ds(base, rows)],
                        shared.at[pl.ds(base, rows)])
        pltpu.sync_copy(idx_hbm.at[wid], idx_v)
        pltpu.sync_copy(ones_hbm, ones_v)
        plsc.subcore_barrier()

        @pl.loop(0, C)
        def _scat(j):
            pltpu.sync_copy(ones_v, shared.at[idx_v.at[j]], add=True)

        plsc.subcore_barrier()
        pltpu.sync_copy(shared.at[pl.ds(base, rows)],
                        out_hbm.at[cid, pl.ds(base, rows)])

    return k(idx3, ones, zeros)


def _tc_embed(x, W1, b1, W2, b2):
    N = x.shape[0]

    def body(x_ref, W1_ref, b1_ref, W2_ref, b2_ref, h_ref, g_ref):
        nf = jnp.dot(x_ref[...], W1_ref[...],
                     preferred_element_type=jnp.float32) + b1_ref[...]
        h = jnp.dot(_lrelu(nf), W2_ref[...],
                    preferred_element_type=jnp.float32) + b2_ref[...]
        h_ref[...] = h
        g_ref[...] = _lrelu(h)

    return pl.pallas_call(
        body,
        out_shape=[jax.ShapeDtypeStruct((N, _D), jnp.float32),
                   jax.ShapeDtypeStruct((N, _D), jnp.float32)],
    )(x, W1, b1.reshape(1, -1), W2, b2.reshape(1, -1))


def _tc_msg(hg8, efT, Wt, bt, A, Bp):
    G = efT.shape[0]
    R = G * Bp

    def body(hg_ref, ef_ref, Wt_ref, bt_ref, A_ref, out_ref):
        Wt = Wt_ref[...]
        bt = bt_ref[...]
        A = A_ref[...]
        hgT = jnp.transpose(hg_ref[...])
        rows = []
        for j in range(8):
            QTj = jnp.dot(Wt, ef_ref[0, j],
                          preferred_element_type=jnp.float32) + bt
            HTj = jnp.dot(A, hgT[j * _D:(j + 1) * _D, :],
                          preferred_element_type=jnp.float32)
            prod = QTj * HTj
            m = prod[0:_D, :]
            for i in range(1, _D):
                m = m + prod[i * _D:(i + 1) * _D, :]
            rows.append(m)
        out_ref[...] = jnp.transpose(jnp.concatenate(rows, axis=0))

    return pl.pallas_call(
        body,
        grid=(G,),
        in_specs=[
            pl.BlockSpec((Bp, 128), lambda i: (i, 0)),
            pl.BlockSpec((1, 8, _D, Bp), lambda i: (i, 0, 0, 0)),
            pl.BlockSpec((_D * _D, _D), lambda i: (0, 0)),
            pl.BlockSpec((_D * _D, 1), lambda i: (0, 0)),
            pl.BlockSpec((_D * _D, _D), lambda i: (0, 0)),
        ],
        out_specs=pl.BlockSpec((Bp, 128), lambda i: (i, 0)),
        out_shape=jax.ShapeDtypeStruct((R, 128), jnp.float32),
    )(hg8, efT, Wt, bt, A)


def _w2_pack(Wef, bef):
    return Wef.T, bef.reshape(_D * _D, 1)


def _a_repl():
    rows = jnp.arange(_D * _D) // _D
    return (rows[:, None] == jnp.arange(_D)[None, :]).astype(jnp.float32)


def _tc_update(p0, p1, c0, c1, b_nn, W_out, b_out):
    N = p0.shape[0]

    def body(p0_ref, p1_ref, c0_ref, c1_ref, bn_ref, W_ref, bo_ref,
             h_ref, g_ref):
        s = p0_ref[...] + p1_ref[...]
        cnt = jnp.maximum(c0_ref[...] + c1_ref[...], 1.0)
        agg = s / cnt + bn_ref[...]
        h = jnp.dot(_lrelu(agg), W_ref[...],
                    preferred_element_type=jnp.float32) + bo_ref[...]
        h_ref[...] = h
        g_ref[...] = _lrelu(h)

    return pl.pallas_call(
        body,
        out_shape=[jax.ShapeDtypeStruct((N, _D), jnp.float32),
                   jax.ShapeDtypeStruct((N, _D), jnp.float32)],
    )(p0, p1, c0, c1, b_nn.reshape(1, -1), W_out, b_out.reshape(1, -1))


def _tc_interaction(lig_h, rec_h):
    L, R = lig_h.shape[0], rec_h.shape[0]
    BL = 256
    grid = (L // BL,)
    inv = 1.0 / (L * R)

    def body(l_ref, r_ref, op_ref, s_ref):
        i = pl.program_id(0)
        op_ref[...] = lax.dot_general(
            l_ref[...], r_ref[...], (((1,), (1,)), ((), ())),
            preferred_element_type=jnp.float32)
        sl = jnp.sum(l_ref[...], axis=0, keepdims=True)
        sr = jnp.sum(r_ref[...], axis=0, keepdims=True)
        part = jnp.sum(sl * sr) * inv

        @pl.when(i == 0)
        def _():
            s_ref[...] = jnp.zeros_like(s_ref)

        s_ref[...] = s_ref[...] + part

    op, s8 = pl.pallas_call(
        body,
        grid=grid,
        in_specs=[pl.BlockSpec((BL, _D), lambda i: (i, 0)),
                  pl.BlockSpec((R, _D), lambda i: (0, 0))],
        out_specs=[pl.BlockSpec((BL, R), lambda i: (i, 0)),
                   pl.BlockSpec((8, 128), lambda i: (0, 0))],
        out_shape=[jax.ShapeDtypeStruct((L, R), jnp.float32),
                   jax.ShapeDtypeStruct((8, 128), jnp.float32)],
    )(lig_h, rec_h)
    return op, s8[0, 0].reshape(1)


def _graph_chain(x, eattr, eidx, W_emb, b_emb, W_hid, b_hid,
                 W_ef, b_ef, b_nn, W_out, b_out, N, E, C, Lc, Bp):
    src = eidx[0].astype(jnp.int32).reshape(_NW, C, Lc)
    dst = eidx[1].astype(jnp.int32).reshape(_NW, C, Lc)
    zeros = jnp.zeros((N, _D), jnp.float32)
    ones = jnp.ones((Lc, _D), jnp.float32)

    efT = eattr.reshape(E // 8 // Bp, Bp, 8, _D).transpose(0, 2, 3, 1)
    h, g = _tc_embed(x, W_emb, b_emb, W_hid, b_hid)
    cparts = _sc_count(dst, ones, zeros, N, C, Lc)
    c0, c1 = cparts[0], cparts[1]

    for layer in range(W_ef.shape[0]):
        hg = _sc_gather(g, src, E, C, Lc)
        hg8 = hg.reshape(E // 8, 128)
        Wt, bt = _w2_pack(W_ef[layer], b_ef[layer])
        msg8 = _tc_msg(hg8, efT, Wt, bt, _a_repl(), Bp)
        msg = msg8.reshape(E, _D)
        parts = _sc_scatter(msg, dst, zeros, N, C, Lc)
        h, g = _tc_update(parts[0], parts[1], c0, c1,
                          b_nn[layer], W_out[layer], b_out[layer])
    return h


def kernel(lig_x, rec_x, lig_edge_attr, rec_edge_attr, lig_edge_index,
           rec_edge_index, W_emb_lig, b_emb_lig, W_emb_rec, b_emb_rec,
           W_hid_lig, b_hid_lig, W_hid_rec, b_hid_rec,
           lig_W_ef, lig_b_ef, lig_b_nn, lig_W_out, lig_b_out,
           rec_W_ef, rec_b_ef, rec_b_nn, rec_W_out, rec_b_out):
    lig_h = _graph_chain(lig_x, lig_edge_attr, lig_edge_index,
                         W_emb_lig, b_emb_lig, W_hid_lig, b_hid_lig,
                         lig_W_ef, lig_b_ef, lig_b_nn, lig_W_out, lig_b_out,
                         _LIG_N, _LIG_E, C=8, Lc=128, Bp=2048)
    rec_h = _graph_chain(rec_x, rec_edge_attr, rec_edge_index,
                         W_emb_rec, b_emb_rec, W_hid_rec, b_hid_rec,
                         rec_W_ef, rec_b_ef, rec_b_nn, rec_W_out, rec_b_out,
                         _REC_N, _REC_E, C=40, Lc=125, Bp=2000)
    op, out = _tc_interaction(lig_h, rec_h)
    return (out, op)
